# baseline (speedup 1.0000x reference)
import functools

import jax
import jax.numpy as jnp
from jax import lax
from jax.experimental import pallas as pl
from jax.experimental.pallas import tpu as pltpu
from jax.experimental.pallas import tpu_sc as plsc

GENRES = 100000
CHANNELS = 32
BATCH = 16384

NUM_CORES = 2
NUM_SUBCORES = 16

CHUNK = 4096
NCHUNK = BATCH // CHUNK
ROWS = CHUNK // 128


def _embed_body(genre_hbm, table_hbm, out_hbm, chan_v, idx_v, out_v,
                chan_sem, idx_sem, out_sem):
    ch = lax.axis_index("c") * NUM_SUBCORES + lax.axis_index("s")
    chan_cp = pltpu.async_copy(table_hbm.at[ch], chan_v, chan_sem)
    idx_cp = pltpu.async_copy(
        genre_hbm.at[pl.ds(0, ROWS), :], idx_v.at[pl.ds(0, ROWS), :], idx_sem
    )
    chan_cp.wait()
    out_cps = []
    for q in range(NCHUNK):
        if q + 1 < NCHUNK:
            next_idx_cp = pltpu.async_copy(
                genre_hbm.at[pl.ds((q + 1) * ROWS, ROWS), :],
                idx_v.at[pl.ds(((q + 1) % 2) * ROWS, ROWS), :],
                idx_sem,
            )
        idx_cp.wait()
        if q >= 2:
            out_cps[q - 2].wait()
        ibase = (q % 2) * ROWS

        @plsc.parallel_loop(0, ROWS, unroll=4)
        def row_body(r, ibase=ibase):
            for k in range(8):
                g = idx_v[ibase + r, pl.ds(k * 16, 16)]
                out_v[ibase + r, pl.ds(k * 16, 16)] = plsc.load_gather(
                    chan_v, [g]
                )
        out_cps.append(
            pltpu.async_copy(
                out_v.at[pl.ds(ibase, ROWS), :],
                out_hbm.at[pl.ds(ch * 128 + q * ROWS, ROWS)],
                out_sem,
            )
        )
        if q + 1 < NCHUNK:
            idx_cp = next_idx_cp
    out_cps[-2].wait()
    out_cps[-1].wait()


@jax.jit
def _lookup(genre, table):
    mesh = plsc.VectorSubcoreMesh(core_axis_name="c", subcore_axis_name="s")
    out = pl.kernel(
        _embed_body,
        out_type=jax.ShapeDtypeStruct((BATCH * CHANNELS // 128, 128),
                                      jnp.float32),
        mesh=mesh,
        scratch_types=[
            pltpu.VMEM((GENRES,), jnp.float32),
            pltpu.VMEM((2 * ROWS, 128), jnp.int32),
            pltpu.VMEM((2 * ROWS, 128), jnp.float32),
            pltpu.SemaphoreType.DMA,
            pltpu.SemaphoreType.DMA,
            pltpu.SemaphoreType.DMA,
        ],
        compiler_params=pltpu.CompilerParams(
            use_tc_tiling_on_sc=True, needs_layout_passes=False
        ),
    )(genre.reshape(BATCH // 128, 128), table.T)
    return out.reshape(CHANNELS, 1, BATCH).transpose(2, 0, 1).reshape(
        BATCH, CHANNELS, 1, 1)


def kernel(genre, table):
    return _lookup(genre, table)


# speedup vs baseline: 1.1866x; 1.1866x over previous
import functools

import jax
import jax.numpy as jnp
from jax import lax
from jax.experimental import pallas as pl
from jax.experimental.pallas import tpu as pltpu
from jax.experimental.pallas import tpu_sc as plsc

GENRES = 100000
CHANNELS = 32
BATCH = 16384

NUM_CORES = 2
NUM_SUBCORES = 16

CHUNK = 4096
NCHUNK = BATCH // CHUNK
ROWS = CHUNK // 128
IDX_ROWS = BATCH // 128


def _embed_body(genre_hbm, table_hbm, out_hbm, chan_v, idx_v, out_v, idx_sh,
                chan_sem, idx_sem, out_sem):
    s = lax.axis_index("s")
    ch = lax.axis_index("c") * NUM_SUBCORES + s
    chan_cp = pltpu.async_copy(table_hbm.at[ch], chan_v, chan_sem)

    @pl.when(s == 0)
    def _():
        pltpu.async_copy(genre_hbm, idx_sh, idx_sem).wait()

    plsc.subcore_barrier()
    idx_cp = pltpu.async_copy(idx_sh, idx_v, idx_sem)

    chan_cp.wait()
    idx_cp.wait()
    out_cps = []
    for q in range(NCHUNK):
        if q >= 2:
            out_cps[q - 2].wait()
        obase = (q % 2) * ROWS
        qbase = q * ROWS

        @plsc.parallel_loop(0, ROWS, unroll=2)
        def row_body(r, obase=obase, qbase=qbase):
            for k in range(8):
                g = idx_v[qbase + r, pl.ds(k * 16, 16)]
                out_v[obase + r, pl.ds(k * 16, 16)] = plsc.load_gather(
                    chan_v, [g]
                )

        out_cps.append(
            pltpu.async_copy(
                out_v.at[pl.ds(obase, ROWS), :],
                out_hbm.at[pl.ds(ch * 128 + q * ROWS, ROWS)],
                out_sem,
            )
        )
    out_cps[-2].wait()
    out_cps[-1].wait()


@jax.jit
def _lookup(genre, table):
    mesh = plsc.VectorSubcoreMesh(core_axis_name="c", subcore_axis_name="s")
    out = pl.kernel(
        _embed_body,
        out_type=jax.ShapeDtypeStruct((BATCH * CHANNELS // 128, 128),
                                      jnp.float32),
        mesh=mesh,
        scratch_types=[
            pltpu.VMEM((GENRES,), jnp.float32),
            pltpu.VMEM((IDX_ROWS, 128), jnp.int32),
            pltpu.VMEM((2 * ROWS, 128), jnp.float32),
            pltpu.VMEM_SHARED((IDX_ROWS, 128), jnp.int32),
            pltpu.SemaphoreType.DMA,
            pltpu.SemaphoreType.DMA,
            pltpu.SemaphoreType.DMA,
        ],
        compiler_params=pltpu.CompilerParams(
            use_tc_tiling_on_sc=True, needs_layout_passes=False
        ),
    )(genre.reshape(BATCH // 128, 128), table.T)
    return out.reshape(CHANNELS, 1, BATCH).transpose(2, 0, 1).reshape(
        BATCH, CHANNELS, 1, 1)


def kernel(genre, table):
    return _lookup(genre, table)


# speedup vs baseline: 1.1908x; 1.0035x over previous
import functools

import jax
import jax.numpy as jnp
from jax import lax
from jax.experimental import pallas as pl
from jax.experimental.pallas import tpu as pltpu
from jax.experimental.pallas import tpu_sc as plsc

GENRES = 100000
CHANNELS = 32
BATCH = 16384

NUM_CORES = 2
NUM_SUBCORES = 16

CHUNK = 4096
NCHUNK = BATCH // CHUNK
ROWS = CHUNK // 128
IDX_ROWS = BATCH // 128


def _embed_body(genre_hbm, table_hbm, out_hbm, chan_v, idx_v, out_v, idx_sh,
                chan_sem, idx_sem, out_sem):
    s = lax.axis_index("s")
    ch = lax.axis_index("c") * NUM_SUBCORES + s
    chan_cp = pltpu.async_copy(table_hbm.at[ch], chan_v, chan_sem)

    @pl.when(s == 0)
    def _():
        pltpu.async_copy(genre_hbm, idx_sh, idx_sem).wait()

    plsc.subcore_barrier()
    idx_cp = pltpu.async_copy(idx_sh, idx_v, idx_sem)

    chan_cp.wait()
    idx_cp.wait()
    out_cps = []
    for q in range(NCHUNK):
        if q >= 2:
            out_cps[q - 2].wait()
        obase = (q % 2) * ROWS
        qbase = q * ROWS

        @plsc.parallel_loop(0, ROWS, unroll=2)
        def row_body(r, obase=obase, qbase=qbase):
            for k in range(8):
                g = idx_v[qbase + r, pl.ds(k * 16, 16)]
                out_v[obase + r, pl.ds(k * 16, 16)] = plsc.load_gather(
                    chan_v, [g]
                )

        out_cps.append(
            pltpu.async_copy(
                out_v.at[pl.ds(obase, ROWS), :],
                out_hbm.at[pl.ds(ch * 128 + q * ROWS, ROWS)],
                out_sem,
            )
        )
    out_cps[-2].wait()
    out_cps[-1].wait()


@jax.jit
def _lookup(genre, table):
    mesh = plsc.VectorSubcoreMesh(core_axis_name="c", subcore_axis_name="s")
    out = pl.kernel(
        _embed_body,
        out_type=jax.ShapeDtypeStruct((BATCH * CHANNELS // 128, 128),
                                      jnp.float32),
        mesh=mesh,
        scratch_types=[
            pltpu.VMEM((GENRES,), jnp.float32),
            pltpu.VMEM((IDX_ROWS, 128), jnp.int32),
            pltpu.VMEM((2 * ROWS, 128), jnp.float32),
            pltpu.VMEM_SHARED((IDX_ROWS, 128), jnp.int32),
            pltpu.SemaphoreType.DMA,
            pltpu.SemaphoreType.DMA,
            pltpu.SemaphoreType.DMA,
        ],
        compiler_params=pltpu.CompilerParams(
            use_tc_tiling_on_sc=True,
            needs_layout_passes=False,
            skip_device_barrier=True,
        ),
    )(genre.reshape(BATCH // 128, 128), table.T)
    return out.reshape(CHANNELS, 1, BATCH).transpose(2, 0, 1).reshape(
        BATCH, CHANNELS, 1, 1)


def kernel(genre, table):
    return _lookup(genre, table)
